# Initial kernel scaffold; baseline (speedup 1.0000x reference)
import jax
import jax.numpy as jnp
from jax.experimental import pallas as pl


def kernel(x, edge_index, W1l, W1r, b1, g1, bb1, W2l, W2r, b2, g2, bb2, W3l, W3r, b3, g3, bb3, W4l, W4r, b4, g4, bb4, Wal, War, ba, Wsl, Wsr, bs, Wel, Wer, be, Wrl, Wrr, br, Wml, Wmr, bm):
    raise NotImplementedError("write your pallas kernel here")


# speedup vs baseline: 3.7241x; 3.7241x over previous
import jax
import jax.numpy as jnp
from jax import lax
from jax.experimental import pallas as pl
from jax.experimental.pallas import tpu as pltpu
from jax.experimental.pallas import tpu_sc as plsc

_N = 10000
_E = 320000
_D = 128
_NC = 2
_NS = 16
_NW = _NC * _NS
_EPW = _E // _NW
_K = 80
_ITERS = _EPW // _K
_NP = 10240
_RPT = _NP // _NS
_CW = 16
_EPS = 1e-5
_R = 2000
_G = _N // _R
_HO = 25


def _seg_body(h_hbm, src_hbm, dst_hbm, znd_hbm,
              agg_out,
              src_v, dst_v, rows_v, sem, agg_sh):
    c = lax.axis_index("c")
    s = lax.axis_index("s")
    wid = c * _NS + s
    r0 = s * _RPT
    pltpu.sync_copy(znd_hbm, rows_v)
    for j in range(_RPT // _K):
        pltpu.sync_copy(rows_v, agg_sh.at[pl.ds(r0 + j * _K, _K)])
    plsc.subcore_barrier()

    def step(i, carry):
        base = wid * _EPW + i * _K
        pltpu.sync_copy(src_hbm.at[pl.ds(base, _K)], src_v)
        pltpu.sync_copy(dst_hbm.at[pl.ds(base, _K)], dst_v)
        pltpu.async_copy(h_hbm.at[src_v], rows_v, sem).wait()
        pltpu.sync_copy(rows_v, agg_sh.at[dst_v], add=True)
        return carry

    lax.fori_loop(0, _ITERS, step, 0)
    plsc.subcore_barrier()
    for j in range(_RPT // _K):
        pltpu.sync_copy(agg_sh.at[pl.ds(r0 + j * _K, _K)], rows_v)
        pltpu.sync_copy(rows_v, agg_out.at[pl.ds(c * _NP + r0 + j * _K, _K)])


_SEG_CACHE = {}


def _seg_kernels():
    if "k" not in _SEG_CACHE:
        mesh = plsc.VectorSubcoreMesh(
            core_axis_name="c", subcore_axis_name="s",
            num_cores=_NC, num_subcores=_NS,
        )
        seg = pl.kernel(
            _seg_body,
            out_type=[jax.ShapeDtypeStruct((_NC * _NP, _D), jnp.float32)],
            mesh=mesh,
            scratch_types=[
                pltpu.VMEM((_K,), jnp.int32),
                pltpu.VMEM((_K,), jnp.int32),
                pltpu.VMEM((_K, _D), jnp.float32),
                pltpu.SemaphoreType.DMA,
                pltpu.VMEM_SHARED((_NP, _D), jnp.float32),
            ],
        )
        _SEG_CACHE["k"] = (seg,)
    return _SEG_CACHE["k"]


def _dot(a, b):
    return jnp.dot(a, b, precision=lax.Precision.HIGHEST,
                   preferred_element_type=jnp.float32)


def _z_and_stats(i, agg, inv, h, wl, wr, b, z_ref, sum_ref, sq_ref):
    z = _dot(agg * inv, wl) + _dot(h, wr) + b
    z_ref[...] = z

    @pl.when(i == 0)
    def _():
        sum_ref[...] = jnp.zeros_like(sum_ref)
        sq_ref[...] = jnp.zeros_like(sq_ref)

    sum_ref[...] += jnp.sum(z, axis=0, keepdims=True)
    sq_ref[...] += jnp.sum(z * z, axis=0, keepdims=True)


def _tcA1_body(a0_ref, a1_ref, c0_ref, c1_ref, h_ref, wl_ref, wr_ref, b_ref,
               z_ref, inv_ref, sum_ref, sq_ref):
    i = pl.program_id(0)
    deg = c0_ref[:, 0:1] + c1_ref[:, 0:1]
    inv = 1.0 / jnp.maximum(deg, 1.0)
    inv_ref[...] = inv
    agg = a0_ref[...] + a1_ref[...]
    _z_and_stats(i, agg, inv, h_ref[...], wl_ref[...], wr_ref[...], b_ref[...],
                 z_ref, sum_ref, sq_ref)


def _tcA_body(a0_ref, a1_ref, inv_ref, h_ref, wl_ref, wr_ref, b_ref,
              z_ref, sum_ref, sq_ref):
    i = pl.program_id(0)
    agg = a0_ref[...] + a1_ref[...]
    _z_and_stats(i, agg, inv_ref[...], h_ref[...], wl_ref[...], wr_ref[...],
                 b_ref[...], z_ref, sum_ref, sq_ref)


def _tcB_body(z_ref, sum_ref, sq_ref, g_ref, bb_ref, out_ref):
    mu = sum_ref[...] * (1.0 / _N)
    var = sq_ref[...] * (1.0 / _N) - mu * mu
    zn = (z_ref[...] - mu) * lax.rsqrt(var + _EPS)
    out_ref[...] = jnp.maximum(zn * g_ref[...] + bb_ref[...], 0.0)


def _tch_body(a0_ref, a1_ref, inv_ref, h_ref, wl_ref, wr_ref, b_ref, out_ref):
    agg = a0_ref[...] + a1_ref[...]
    mean = agg * inv_ref[...]
    out_ref[...] = _dot(mean, wl_ref[...]) + _dot(h_ref[...], wr_ref[...]) \
        + b_ref[...]


def _rows(i):
    return (i, 0)


def _rep(i):
    return (0, 0)


_agg_spec = pl.BlockSpec((_R, _D), _rows)
_cnt_spec = pl.BlockSpec((_R, _CW), _rows)
_inv_spec = pl.BlockSpec((_R, 1), _rows)
_w_spec = pl.BlockSpec((_D, _D), _rep)
_b_spec = pl.BlockSpec((1, _D), _rep)
_stat_spec = pl.BlockSpec((1, _D), _rep)

_tcA1_call = pl.pallas_call(
    _tcA1_body,
    grid=(_G,),
    in_specs=[_agg_spec, _agg_spec, _cnt_spec, _cnt_spec, _agg_spec,
              _w_spec, _w_spec, _b_spec],
    out_specs=(_agg_spec, _inv_spec, _stat_spec, _stat_spec),
    out_shape=(
        jax.ShapeDtypeStruct((_N, _D), jnp.float32),
        jax.ShapeDtypeStruct((_N, 1), jnp.float32),
        jax.ShapeDtypeStruct((1, _D), jnp.float32),
        jax.ShapeDtypeStruct((1, _D), jnp.float32),
    ),
)

_tcA_call = pl.pallas_call(
    _tcA_body,
    grid=(_G,),
    in_specs=[_agg_spec, _agg_spec, _inv_spec, _agg_spec,
              _w_spec, _w_spec, _b_spec],
    out_specs=(_agg_spec, _stat_spec, _stat_spec),
    out_shape=(
        jax.ShapeDtypeStruct((_N, _D), jnp.float32),
        jax.ShapeDtypeStruct((1, _D), jnp.float32),
        jax.ShapeDtypeStruct((1, _D), jnp.float32),
    ),
)

_tcB_call = pl.pallas_call(
    _tcB_body,
    grid=(_G,),
    in_specs=[_agg_spec, _stat_spec, _stat_spec, _b_spec, _b_spec],
    out_specs=_agg_spec,
    out_shape=jax.ShapeDtypeStruct((_N, _D), jnp.float32),
)

_tch_call = pl.pallas_call(
    _tch_body,
    grid=(_G,),
    in_specs=[_agg_spec, _agg_spec, _inv_spec, _agg_spec,
              pl.BlockSpec((_D, _HO), _rep), pl.BlockSpec((_D, _HO), _rep),
              pl.BlockSpec((1, _HO), _rep)],
    out_specs=pl.BlockSpec((_R, _HO), _rows),
    out_shape=jax.ShapeDtypeStruct((_N, _HO), jnp.float32),
)


def kernel(x, edge_index, W1l, W1r, b1, g1, bb1, W2l, W2r, b2, g2, bb2,
           W3l, W3r, b3, g3, bb3, W4l, W4r, b4, g4, bb4, Wal, War, ba,
           Wsl, Wsr, bs, Wel, Wer, be, Wrl, Wrr, br, Wml, Wmr, bm):
    src = edge_index[0]
    dst = edge_index[1]
    znd = jnp.zeros((_K, _D), jnp.float32)
    ones_nd = jnp.ones((_N, _D), jnp.float32)
    (_seg_call,) = _seg_kernels()

    cnt_f = _seg_call(ones_nd, src, dst, znd)[0][:, 0:_CW]
    agg_f = _seg_call(x, src, dst, znd)[0]
    a0, a1 = agg_f[0:_NP], agg_f[_NP:2 * _NP]
    c0, c1 = cnt_f[0:_NP], cnt_f[_NP:2 * _NP]
    z, inv, s1, s2 = _tcA1_call(a0, a1, c0, c1, x, W1l.T, W1r.T,
                                b1.reshape(1, _D))
    h = _tcB_call(z, s1, s2, g1.reshape(1, _D), bb1.reshape(1, _D))

    for (Wl, Wr, bc, g, bb) in ((W2l, W2r, b2, g2, bb2),
                                (W3l, W3r, b3, g3, bb3),
                                (W4l, W4r, b4, g4, bb4)):
        agg_f = _seg_call(h, src, dst, znd)[0]
        a0, a1 = agg_f[0:_NP], agg_f[_NP:2 * _NP]
        z, s1, s2 = _tcA_call(a0, a1, inv, h, Wl.T, Wr.T, bc.reshape(1, _D))
        h = _tcB_call(z, s1, s2, g.reshape(1, _D), bb.reshape(1, _D))

    agg_f = _seg_call(h, src, dst, znd)[0]
    a0, a1 = agg_f[0:_NP], agg_f[_NP:2 * _NP]
    wl_cat = jnp.concatenate([Wal, Wsl, Wel, Wrl, Wml], axis=0).T
    wr_cat = jnp.concatenate([War, Wsr, Wer, Wrr, Wmr], axis=0).T
    b_cat = jnp.concatenate([ba, bs, be, br, bm]).reshape(1, _HO)
    out = _tch_call(a0, a1, inv, h, wl_cat, wr_cat, b_cat)
    return (out[:, 0:3], out[:, 3:5], out[:, 5:10], out[:, 10:19],
            out[:, 19:25])
